# Initial kernel scaffold; baseline (speedup 1.0000x reference)
import jax
import jax.numpy as jnp
from jax.experimental import pallas as pl


def kernel(input, mask_samples):
    raise NotImplementedError("write your pallas kernel here")


# speedup vs baseline: 4.7081x; 4.7081x over previous
import jax
import jax.numpy as jnp
from jax.experimental import pallas as pl

_M = 8192
_N = 8192
_FX = 1000.0 / 640.0
_FY = 1000.0 / 480.0
_ZOFF = 2.5
_CH = 256


def _chamfer_body(inp_ref, yT_ref, out_ref):
    yx = yT_ref[0:1, :]
    yy = yT_ref[1:2, :]

    def body(i, carry):
        rowsum, colmin = carry
        chunk = inp_ref[pl.ds(i * _CH, _CH), :]
        z = chunk[:, 2:3] + _ZOFF
        px = chunk[:, 0:1] * _FX / z
        py = chunk[:, 1:2] * _FY / z
        dx = px - yx
        dy = py - yy
        d2 = dx * dx + dy * dy
        rowsum = rowsum + jnp.sum(jnp.min(d2, axis=1, keepdims=True))
        colmin = jnp.minimum(colmin, jnp.min(d2, axis=0, keepdims=True))
        return rowsum, colmin

    init = (jnp.float32(0.0), jnp.full((1, _N), jnp.inf, dtype=jnp.float32))
    rowsum, colmin = jax.lax.fori_loop(0, _M // _CH, body, init)
    out_ref[...] = jnp.reshape(rowsum / _M + jnp.sum(colmin) / _N, (1, 1))


@jax.jit
def kernel(input, mask_samples):
    yT = mask_samples[0].T
    out = pl.pallas_call(
        _chamfer_body,
        out_shape=jax.ShapeDtypeStruct((1, 1), jnp.float32),
    )(input, yT)
    return out[0, 0]


# speedup vs baseline: 8.3364x; 1.7707x over previous
import jax
import jax.numpy as jnp
from jax.experimental import pallas as pl

_M = 8192
_N = 8192
_FX = 1000.0 / 640.0
_FY = 1000.0 / 480.0
_ZOFF = 2.5
_CH = 256


def _chamfer_body(inp_ref, yT_ref, out_ref):
    yT = yT_ref[...]
    yx = yT[0:1, :]
    yy = yT[1:2, :]
    s = yx * yx + yy * yy

    def body(i, carry):
        rowsum, colmin = carry
        chunk = inp_ref[pl.ds(i * _CH, _CH), :]
        z = chunk[:, 2:3] + _ZOFF
        px = chunk[:, 0:1] * _FX / z
        py = chunk[:, 1:2] * _FY / z
        r = px * px + py * py
        pmat = jnp.concatenate([px + px, py + py], axis=1)
        g = jax.lax.dot_general(
            pmat, yT, (((1,), (0,)), ((), ())),
            preferred_element_type=jnp.float32)
        d2 = (s - g) + r
        rowsum = rowsum + jnp.sum(jnp.min(d2, axis=1, keepdims=True))
        colmin = jnp.minimum(colmin, jnp.min(d2, axis=0, keepdims=True))
        return rowsum, colmin

    init = (jnp.float32(0.0), jnp.full((1, _N), jnp.inf, dtype=jnp.float32))
    rowsum, colmin = jax.lax.fori_loop(0, _M // _CH, body, init)
    out_ref[...] = jnp.reshape(rowsum / _M + jnp.sum(colmin) / _N, (1, 1))


@jax.jit
def kernel(input, mask_samples):
    yT = mask_samples[0].T
    out = pl.pallas_call(
        _chamfer_body,
        out_shape=jax.ShapeDtypeStruct((1, 1), jnp.float32),
    )(input, yT)
    return out[0, 0]
